# baseline (speedup 1.0000x reference)
import functools

import jax
import jax.numpy as jnp
from jax import lax
from jax.experimental import pallas as pl

K = 16
NEG_INF = float("-inf")


def _gelu(v):
    return 0.5 * v * (1.0 + lax.erf(v * 0.7071067811865476))


def _local_block(p, f, W1, b1, W2, b2, iota_l, iota_big, mlp_prec):
    N = p.shape[0]
    F = f.shape[1]
    HI = lax.Precision.HIGHEST
    pp = p * p
    rcol = jnp.sum(pp, axis=1, keepdims=True)
    rrow = jnp.reshape(jnp.sum(pp, axis=1), (1, N))
    m = lax.dot_general(p, p, (((1,), (1,)), ((), ())),
                        precision=HI)
    negD = -(rcol - 2.0 * m + rrow + 1e-05)

    nd = negD
    sels = []
    for _ in range(K + 1):
        mx = jnp.max(nd, axis=1, keepdims=True)
        sel = jnp.min(jnp.where(nd == mx, iota_l, N), axis=1, keepdims=True)
        sels.append(sel)
        nd = jnp.where(iota_l == sel, NEG_INF, nd)

    W1a = W1[:F, :]
    W1d = W1[F:, :] - W1a
    cterm = lax.dot_general(f, W1d, (((1,), (0,)), ((), ())),
                            precision=mlp_prec) + b1

    sel_stack = jnp.concatenate(sels[1:], axis=0)
    oh = (iota_big == sel_stack).astype(jnp.float32)
    g = lax.dot_general(oh, f, (((1,), (0,)), ((), ())),
                        precision=mlp_prec)
    cbig = jnp.concatenate([cterm] * K, axis=0)
    h1 = _gelu(lax.dot_general(g, W1a, (((1,), (0,)), ((), ())),
                               precision=mlp_prec) + cbig)
    h2 = _gelu(lax.dot_general(h1, W2, (((1,), (0,)), ((), ())),
                               precision=mlp_prec) + b2)
    acc = None
    for k in range(K):
        part = h2[k * N:(k + 1) * N, :]
        acc = part if acc is None else acc + part
    return acc * (1.0 / K)


def _body(x_ref, pts_ref, mask_ref, W01_ref, b01_ref, W02_ref, b02_ref,
          W11_ref, b11_ref, W12_ref, b12_ref, out_ref):
    f = x_ref[0]
    pts = pts_ref[0]
    mcol = mask_ref[0]
    N = f.shape[0]
    shift = 999.0 * (mcol == 0.0).astype(jnp.float32)
    iota_l = lax.broadcasted_iota(jnp.int32, (N, N), 1)
    iota_big = lax.broadcasted_iota(jnp.int32, (K * N, N), 1)
    HI = lax.Precision.HIGHEST

    f1 = _local_block(shift + pts, f, W01_ref[...], b01_ref[...],
                      W02_ref[...], b02_ref[...], iota_l, iota_big, HI)
    f2 = _local_block(shift + f1, f1, W11_ref[...], b11_ref[...],
                      W12_ref[...], b12_ref[...], iota_l, iota_big, HI)
    out_ref[0] = f2 * mcol


@functools.partial(jax.jit, static_argnames=())
def kernel(x, points, mask, W01, b01, W02, b02, W11, b11, W12, b12):
    B, N, F = x.shape
    P = W02.shape[1]
    mask_f = mask.astype(jnp.float32)
    b01r = b01.reshape(1, -1)
    b02r = b02.reshape(1, -1)
    b11r = b11.reshape(1, -1)
    b12r = b12.reshape(1, -1)

    full = lambda s: pl.BlockSpec(s, lambda b: (0,) * len(s))
    out = pl.pallas_call(
        _body,
        grid=(B,),
        in_specs=[
            pl.BlockSpec((1, N, F), lambda b: (b, 0, 0)),
            pl.BlockSpec((1, N, 3), lambda b: (b, 0, 0)),
            pl.BlockSpec((1, N, 1), lambda b: (b, 0, 0)),
            full(W01.shape), full(b01r.shape), full(W02.shape), full(b02r.shape),
            full(W11.shape), full(b11r.shape), full(W12.shape), full(b12r.shape),
        ],
        out_specs=pl.BlockSpec((1, N, P), lambda b: (b, 0, 0)),
        out_shape=jax.ShapeDtypeStruct((B, N, P), jnp.float32),
    )(x, points, mask_f, W01, b01r, W02, b02r, W11, b11r, W12, b12r)
    return out


# speedup vs baseline: 2.6392x; 2.6392x over previous
import functools

import jax
import jax.numpy as jnp
from jax import lax
from jax.experimental import pallas as pl

K = 16
NEG_INF = float("-inf")


def _gelu(v):
    return 0.5 * v * (1.0 + lax.erf(v * 0.7071067811865476))


def _local_block(p, f, W1, b1, W2, b2, iota_l, mlp_prec):
    N = p.shape[0]
    F = f.shape[1]
    HI = lax.Precision.HIGHEST
    pp = p * p
    rcol = jnp.sum(pp, axis=1, keepdims=True)
    rrow = jnp.reshape(jnp.sum(pp, axis=1), (1, N))
    m = lax.dot_general(p, p, (((1,), (1,)), ((), ())),
                        precision=HI)
    negD = -(rcol - 2.0 * m + rrow + 1e-05)

    nd = negD
    ohs = []
    for _ in range(K + 1):
        sel = jnp.argmax(nd, axis=1).astype(jnp.int32)[:, None]
        ohb = iota_l == sel
        ohs.append(ohb)
        nd = jnp.where(ohb, NEG_INF, nd)

    W1a = W1[:F, :]
    W1d = W1[F:, :] - W1a
    cterm = lax.dot_general(f, W1d, (((1,), (0,)), ((), ())),
                            precision=mlp_prec) + b1

    acc = None
    for k in range(1, K + 1):
        oh = ohs[k].astype(jnp.float32)
        g = lax.dot_general(oh, f, (((1,), (0,)), ((), ())),
                            precision=mlp_prec)
        h1 = _gelu(lax.dot_general(g, W1a, (((1,), (0,)), ((), ())),
                                   precision=mlp_prec) + cterm)
        h2 = _gelu(lax.dot_general(h1, W2, (((1,), (0,)), ((), ())),
                                   precision=mlp_prec) + b2)
        acc = h2 if acc is None else acc + h2
    return acc * (1.0 / K)


MB = 1


def _body(x_ref, pts_ref, mask_ref, W01_ref, b01_ref, W02_ref, b02_ref,
          W11_ref, b11_ref, W12_ref, b12_ref, out_ref):
    N = x_ref.shape[1]
    iota_l = lax.broadcasted_iota(jnp.int32, (N, N), 1)
    HI = lax.Precision.HIGHEST
    for i in range(MB):
        f = x_ref[i]
        pts = pts_ref[i]
        mcol = mask_ref[i]
        shift = 999.0 * (mcol == 0.0).astype(jnp.float32)
        f1 = _local_block(shift + pts, f, W01_ref[...], b01_ref[...],
                          W02_ref[...], b02_ref[...], iota_l, HI)
        f2 = _local_block(shift + f1, f1, W11_ref[...], b11_ref[...],
                          W12_ref[...], b12_ref[...], iota_l,
                          lax.Precision.DEFAULT)
        out_ref[i] = f2 * mcol


@functools.partial(jax.jit, static_argnames=())
def kernel(x, points, mask, W01, b01, W02, b02, W11, b11, W12, b12):
    B, N, F = x.shape
    P = W02.shape[1]
    mask_f = mask.astype(jnp.float32)
    b01r = b01.reshape(1, -1)
    b02r = b02.reshape(1, -1)
    b11r = b11.reshape(1, -1)
    b12r = b12.reshape(1, -1)

    full = lambda s: pl.BlockSpec(s, lambda b: (0,) * len(s))
    out = pl.pallas_call(
        _body,
        grid=(B // MB,),
        in_specs=[
            pl.BlockSpec((MB, N, F), lambda b: (b, 0, 0)),
            pl.BlockSpec((MB, N, 3), lambda b: (b, 0, 0)),
            pl.BlockSpec((MB, N, 1), lambda b: (b, 0, 0)),
            full(W01.shape), full(b01r.shape), full(W02.shape), full(b02r.shape),
            full(W11.shape), full(b11r.shape), full(W12.shape), full(b12r.shape),
        ],
        out_specs=pl.BlockSpec((MB, N, P), lambda b: (b, 0, 0)),
        out_shape=jax.ShapeDtypeStruct((B, N, P), jnp.float32),
    )(x, points, mask_f, W01, b01r, W02, b02r, W11, b11r, W12, b12r)
    return out


# speedup vs baseline: 2.8507x; 1.0801x over previous
import functools

import jax
import jax.numpy as jnp
from jax import lax
from jax.experimental import pallas as pl

K = 16
NEG_INF = float("-inf")


def _gelu(v):
    return 0.5 * v * (1.0 + lax.erf(v * 0.7071067811865476))


def _local_block(p, f, W1, b1, W2, b2, iota_l, mlp_prec):
    N = p.shape[0]
    F = f.shape[1]
    HI = lax.Precision.HIGHEST
    pp = p * p
    rcol = jnp.sum(pp, axis=1, keepdims=True)
    rrow = jnp.reshape(jnp.sum(pp, axis=1), (1, N))
    m = lax.dot_general(p, p, (((1,), (1,)), ((), ())),
                        precision=HI)
    negD = -(rcol - 2.0 * m + rrow + 1e-05)

    nd = negD
    ohs = []
    for _ in range(K + 1):
        mx = jnp.max(nd, axis=1, keepdims=True)
        ohb = nd == mx
        ohs.append(ohb)
        nd = jnp.where(ohb, NEG_INF, nd)

    W1a = W1[:F, :]
    W1d = W1[F:, :] - W1a
    cterm = lax.dot_general(f, W1d, (((1,), (0,)), ((), ())),
                            precision=mlp_prec) + b1

    if mlp_prec is lax.Precision.HIGHEST:
        fa = f.astype(jnp.bfloat16).astype(jnp.float32)
        fr = f - fa
        fb = fr.astype(jnp.bfloat16).astype(jnp.float32)
        fc = fr - fb
        fparts = (fa, fb, fc)
    else:
        fparts = (f,)

    acc = None
    for k in range(1, K + 1):
        oh = ohs[k].astype(jnp.float32)
        g = None
        for part in fparts:
            gp = lax.dot_general(oh, part, (((1,), (0,)), ((), ())),
                                 precision=lax.Precision.DEFAULT)
            g = gp if g is None else g + gp
        h1 = _gelu(lax.dot_general(g, W1a, (((1,), (0,)), ((), ())),
                                   precision=mlp_prec) + cterm)
        h2 = _gelu(lax.dot_general(h1, W2, (((1,), (0,)), ((), ())),
                                   precision=mlp_prec) + b2)
        acc = h2 if acc is None else acc + h2
    return acc * (1.0 / K)


MB = 1


def _body(x_ref, pts_ref, mask_ref, W01_ref, b01_ref, W02_ref, b02_ref,
          W11_ref, b11_ref, W12_ref, b12_ref, out_ref):
    N = x_ref.shape[1]
    iota_l = lax.broadcasted_iota(jnp.int32, (N, N), 1)
    HI = lax.Precision.HIGHEST
    for i in range(MB):
        f = x_ref[i]
        pts = pts_ref[i]
        mcol = mask_ref[i]
        shift = 999.0 * (mcol == 0.0).astype(jnp.float32)
        f1 = _local_block(shift + pts, f, W01_ref[...], b01_ref[...],
                          W02_ref[...], b02_ref[...], iota_l, HI)
        f2 = _local_block(shift + f1, f1, W11_ref[...], b11_ref[...],
                          W12_ref[...], b12_ref[...], iota_l,
                          lax.Precision.DEFAULT)
        out_ref[i] = f2 * mcol


@functools.partial(jax.jit, static_argnames=())
def kernel(x, points, mask, W01, b01, W02, b02, W11, b11, W12, b12):
    B, N, F = x.shape
    P = W02.shape[1]
    mask_f = mask.astype(jnp.float32)
    b01r = b01.reshape(1, -1)
    b02r = b02.reshape(1, -1)
    b11r = b11.reshape(1, -1)
    b12r = b12.reshape(1, -1)

    full = lambda s: pl.BlockSpec(s, lambda b: (0,) * len(s))
    out = pl.pallas_call(
        _body,
        grid=(B // MB,),
        in_specs=[
            pl.BlockSpec((MB, N, F), lambda b: (b, 0, 0)),
            pl.BlockSpec((MB, N, 3), lambda b: (b, 0, 0)),
            pl.BlockSpec((MB, N, 1), lambda b: (b, 0, 0)),
            full(W01.shape), full(b01r.shape), full(W02.shape), full(b02r.shape),
            full(W11.shape), full(b11r.shape), full(W12.shape), full(b12r.shape),
        ],
        out_specs=pl.BlockSpec((MB, N, P), lambda b: (b, 0, 0)),
        out_shape=jax.ShapeDtypeStruct((B, N, P), jnp.float32),
    )(x, points, mask_f, W01, b01r, W02, b02r, W11, b11r, W12, b12r)
    return out
